# baseline (speedup 1.0000x reference)
import math

import jax
import jax.numpy as jnp
from jax.experimental import pallas as pl
from jax.experimental.pallas import tpu as pltpu

NS = 64
T = 60
TP = 64
B = 32
DOB = 4
DPE = 16
DM = NS * DOB
DTR = DM + DPE
NH = 4
DH = DTR // NH
NHID = 512
NCLS = 2
DST = 9
DF = T * DOB
MAXC = 100.0
BN = B * NS
BTP = B * TP
DFIN = DM + DPE + NS

SB = 4
SBN = SB * NS
NBLK = B // SB

_F32 = jnp.float32


def _dotT(a, b):
    return jax.lax.dot_general(a, b, (((1,), (1,)), ((), ())),
                               preferred_element_type=_F32)


def _dotC0(a, b):
    return jax.lax.dot_general(a, b, (((0,), (0,)), ((), ())),
                               preferred_element_type=_F32)


def _dot(a, b):
    return jnp.dot(a, b, preferred_element_type=_F32)


def _blockdiag_mask(n):
    r = jax.lax.broadcasted_iota(jnp.int32, (SB * n, SB * n), 0) // n
    c = jax.lax.broadcasted_iota(jnp.int32, (SB * n, SB * n), 1) // n
    return (r == c).astype(_F32)


def _body(xt2d, emat, ru4, gw, w2t, scst, valid, msel,
          stat, wstat, bstat, ew, eb, ln, f1w, f1b, f2w, f2b,
          wm1, bm1, wm2, bm2,
          logits, dist,
          hn, qs, ks, vs, xs, os_, ffs, a1s, a2s, aggs):
    xr = _dot(xt2d[:, :], emat[:, :])
    rut = ru4[:, :]
    rut = jnp.concatenate([rut, rut, rut, rut, rut, rut, rut, rut], axis=0)
    hn[:, :] = jnp.maximum(xr * rut, 0.0)
    h = hn[:, :]
    qs[:, :DF] = _dot(h, gw[0, :, :])
    ks[:, :DF] = _dot(h, gw[1, :, :])
    vs[:, :DF] = _dot(h, gw[2, :, :])
    scale = 1.0 / math.sqrt(float(DF))
    bm = _blockdiag_mask(NS)

    def l1(i, c):
        base = pl.multiple_of(i * SBN, 8)
        qb = qs[pl.ds(base, SBN), :DF]
        kb = ks[pl.ds(base, SBN), :DF]
        vb = vs[pl.ds(base, SBN), :DF]
        s = _dotT(qb, kb) * scale
        m = jnp.max(s, axis=1, keepdims=True)
        e = jnp.exp(s - m) * bm
        a = e / jnp.sum(e, axis=1, keepdims=True)
        a1s[i, :, :] = a
        hn[pl.ds(base, SBN), :] = _dot(a, vb)
        return c

    jax.lax.fori_loop(0, NBLK, l1, 0, unroll=8)

    h1 = hn[:, :]
    qs[:, :DF] = _dot(h1, gw[3, :, :])
    ks[:, :DF] = _dot(h1, gw[4, :, :])

    def l2(i, c):
        base = pl.multiple_of(i * SBN, 8)
        qb = qs[pl.ds(base, SBN), :DF]
        kb = ks[pl.ds(base, SBN), :DF]
        hb = hn[pl.ds(base, SBN), :]
        s = _dotT(qb, kb) * scale * a1s[i, :, :]
        m = jnp.max(s, axis=1, keepdims=True)
        e = jnp.exp(s - m) * bm
        a = e / jnp.sum(e, axis=1, keepdims=True)
        for j in range(SB):
            a2s[i * SB + j, :, :] = a[j * NS:(j + 1) * NS,
                                      j * NS:(j + 1) * NS]
        g = _dot(a, hb)
        for d in range(DOB):
            md = _dotT(w2t[d, :, :], g)
            for j in range(SB):
                rb = pl.multiple_of(base + j * NS, 8)
                xs[pl.ds(rb, NS), d * NS:(d + 1) * NS] = (
                    md[:, j * NS:(j + 1) * NS])
        return c

    jax.lax.fori_loop(0, NBLK, l2, 0, unroll=8)

    amat = a2s[:, :, :]

    def dloop(i, tot):
        bi = a2s[pl.ds(i, 1), :, :]
        diff = amat - bi
        s2 = jnp.sum(diff * diff, axis=2)
        sj = jnp.sum(s2, axis=1, keepdims=True)
        return tot + jnp.sum(jnp.sqrt(sj + 1e-12))

    tot = jax.lax.fori_loop(0, B, dloop, jnp.zeros((1, 1), _F32), unroll=16)
    dist[:, :] = tot / float(B * B)

    sc = scst[:, :]
    xs[:, DM:DM + DPE // 2] = jnp.sin(sc)
    xs[:, DM + DPE // 2:DTR] = jnp.cos(sc)

    hscale = 1.0 / math.sqrt(float(DH))
    bmt = _blockdiag_mask(TP)

    for l in range(2):
        x = xs[:, :]
        qs[:, :] = _dot(x, ew[4 * l + 0, :, :]) + eb[4 * l + 0:4 * l + 1, :]
        ks[:, :] = _dot(x, ew[4 * l + 1, :, :]) + eb[4 * l + 1:4 * l + 2, :]
        vs[:, :] = _dot(x, ew[4 * l + 2, :, :]) + eb[4 * l + 2:4 * l + 3, :]

        def attn_b(i, c):
            base = pl.multiple_of(i * SB * TP, 8)
            vcol = valid[pl.ds(base, SB * TP), :]
            for hh in range(NH):
                lo = hh * DH
                qh = qs[pl.ds(base, SB * TP), lo:lo + DH]
                kh = ks[pl.ds(base, SB * TP), lo:lo + DH]
                vh = vs[pl.ds(base, SB * TP), lo:lo + DH]
                sT = _dotT(kh, qh) * hscale
                m = jnp.max(sT, axis=0, keepdims=True)
                e = jnp.exp(sT - m) * (vcol * bmt)
                a = e / jnp.sum(e, axis=0, keepdims=True)
                os_[pl.ds(base, SB * TP), lo:lo + DH] = _dotC0(a, vh)
            return c

        jax.lax.fori_loop(0, NBLK, attn_b, 0, unroll=8)

        attn = _dot(os_[:, :], ew[4 * l + 3, :, :]) + eb[4 * l + 3:4 * l + 4, :]
        y = xs[:, :] + attn
        mu = jnp.mean(y, axis=1, keepdims=True)
        var = jnp.mean((y - mu) * (y - mu), axis=1, keepdims=True)
        yn = ((y - mu) * jax.lax.rsqrt(var + 1e-5) * ln[4 * l + 0:4 * l + 1, :]
              + ln[4 * l + 1:4 * l + 2, :])
        ffs[:, :] = jnp.maximum(_dot(yn, f1w[l, :, :]) + f1b[l:l + 1, :], 0.0)
        y2 = yn + _dot(ffs[:, :], f2w[l, :, :]) + f2b[l:l + 1, :]
        mu2 = jnp.mean(y2, axis=1, keepdims=True)
        var2 = jnp.mean((y2 - mu2) * (y2 - mu2), axis=1, keepdims=True)
        xs[:, :] = ((y2 - mu2) * jax.lax.rsqrt(var2 + 1e-5)
                    * ln[4 * l + 2:4 * l + 3, :] + ln[4 * l + 3:4 * l + 4, :])

    aggs[:, :DTR] = _dot(msel[:, :], xs[:, :])
    aggs[:, DTR:DFIN] = _dot(stat[:, :], wstat[:, :]) + bstat[:, :]
    hfin = jnp.maximum(_dot(aggs[:, :], wm1[:, :]) + bm1[:, :], 0.0)
    logits[:, :] = _dot(hfin, wm2[:, :]) + bm2[:, :]


def kernel(src, static, times, lengths, params):
    p = params
    f32 = _F32

    x = src[:, :, :NS]
    xt2d = jnp.transpose(x, (1, 2, 0)).reshape(BN, T)
    ru = jnp.broadcast_to(p['R_u'].reshape(NS, 1, DOB),
                          (NS, T, DOB)).reshape(NS, DF)
    ru4 = jnp.tile(ru, (SB, 1))

    ct = jnp.arange(DF)
    emat = (ct[None, :] // DOB == jnp.arange(T)[:, None]).astype(f32)
    w2t = jnp.pad(p['Wv2'].T.reshape(T, DOB, DF).transpose(1, 0, 2),
                  ((0, 0), (0, TP - T), (0, 0)))

    gw = jnp.stack([p['Wq1'], p['Wk1'], p['Wv1'],
                    p['Wq2'], p['Wk2'], p['Wv2']])

    q256 = jnp.arange(DM)
    perm256 = (q256 % NS) * DOB + q256 // NS
    perm272 = jnp.concatenate([perm256, jnp.arange(DM, DTR)])
    perm336 = jnp.concatenate([perm272, jnp.arange(DTR, DFIN)])

    ew = jnp.stack([
        w for l in range(2) for w in (
            p['aWq%d' % l][perm272, :], p['aWk%d' % l][perm272, :],
            p['aWv%d' % l][perm272, :], p['aWo%d' % l][:, perm272])])
    eb = jnp.stack([
        v for l in range(2) for v in (
            p['abq%d' % l], p['abk%d' % l], p['abv%d' % l],
            p['abo%d' % l][perm272])])
    ln = jnp.stack([
        v[perm272] for l in range(2) for v in (
            p['ln1s%d' % l], p['ln1b%d' % l],
            p['ln2s%d' % l], p['ln2b%d' % l])])
    f1w = jnp.stack([p['fW10'][perm272, :], p['fW11'][perm272, :]])
    f1b = jnp.stack([p['fb10'], p['fb11']])
    f2w = jnp.stack([p['fW20'][:, perm272], p['fW21'][:, perm272]])
    f2b = jnp.stack([p['fb20'][perm272], p['fb21'][perm272]])
    wm1 = p['Wm1'][perm336, :]

    scales = (MAXC ** jnp.linspace(0.0, 1.0, DPE // 2)).reshape(
        1, DPE // 2).astype(f32)
    scst = jnp.pad(times.T, ((0, 0), (0, TP - T))).reshape(BTP, 1) / scales
    ar = jnp.arange(TP)
    validm = (ar[None, :] < lengths[:, None]).astype(f32)
    valid = validm.reshape(BTP, 1)
    keep = validm / (lengths[:, None].astype(f32) + 1.0)
    ident = (jax.lax.broadcasted_iota(jnp.int32, (B, B), 0)
             == jax.lax.broadcasted_iota(jnp.int32, (B, B), 1)).astype(f32)
    msel = (ident[:, :, None] * keep[None, :, :]).reshape(B, BTP)

    def r2(v):
        return v.reshape(1, -1)

    logits, dist = pl.pallas_call(
        _body,
        out_shape=(jax.ShapeDtypeStruct((B, NCLS), f32),
                   jax.ShapeDtypeStruct((1, 1), f32)),
        scratch_shapes=[
            pltpu.VMEM((BN, DF), f32),
            pltpu.VMEM((BTP, DTR), f32),
            pltpu.VMEM((BTP, DTR), f32),
            pltpu.VMEM((BTP, DTR), f32),
            pltpu.VMEM((BTP, DTR), f32),
            pltpu.VMEM((BTP, DTR), f32),
            pltpu.VMEM((BTP, NHID), f32),
            pltpu.VMEM((NBLK, SBN, SBN), f32),
            pltpu.VMEM((B, NS, NS), f32),
            pltpu.VMEM((B, DFIN), f32),
        ],
    )(xt2d, emat, ru4, gw, w2t, scst, valid, msel,
      static, p['W_static'], r2(p['b_static']),
      ew, eb, ln, f1w, f1b, f2w, f2b,
      wm1, r2(p['bm1']), p['Wm2'], r2(p['bm2']))

    return logits, dist[0, 0]


# speedup vs baseline: 1.3506x; 1.3506x over previous
import math

import jax
import jax.numpy as jnp
from jax.experimental import pallas as pl
from jax.experimental.pallas import tpu as pltpu

NS = 64
T = 60
TP = 64
B = 32
DOB = 4
DPE = 16
DM = NS * DOB
DTR = DM + DPE
NH = 4
DH = DTR // NH
NHID = 512
NCLS = 2
DST = 9
DF = T * DOB
MAXC = 100.0
BN = B * NS
BTP = B * TP
DFIN = DM + DPE + NS

_F32 = jnp.float32


def _dotT(a, b):
    return jax.lax.dot_general(a, b, (((1,), (1,)), ((), ())),
                               preferred_element_type=_F32)


def _dotC0(a, b):
    return jax.lax.dot_general(a, b, (((0,), (0,)), ((), ())),
                               preferred_element_type=_F32)


def _dot(a, b):
    return jnp.dot(a, b, preferred_element_type=_F32)


SB = 4
SBN = SB * NS
NBLK = B // SB


def _blockdiag_mask(n):
    r = jax.lax.broadcasted_iota(jnp.int32, (SB * n, SB * n), 0) // n
    c = jax.lax.broadcasted_iota(jnp.int32, (SB * n, SB * n), 1) // n
    return (r == c).astype(_F32)


def _gat_body(xt2d, emat, ru4, gw,
              hout, dist, hn, qs, ks, vs, a1s, a2s):
    xr = _dot(xt2d[:, :], emat[:, :])
    rut = ru4[:, :]
    rut = jnp.concatenate([rut] * NBLK, axis=0)
    hn[:, :] = jnp.maximum(xr * rut, 0.0)
    h = hn[:, :]
    qs[:, :] = _dot(h, gw[0, :, :])
    ks[:, :] = _dot(h, gw[1, :, :])
    vs[:, :] = _dot(h, gw[2, :, :])
    scale = 1.0 / math.sqrt(float(DF))
    bm = _blockdiag_mask(NS)

    def l1(i, c):
        base = pl.multiple_of(i * SBN, 8)
        qb = qs[pl.ds(base, SBN), :]
        kb = ks[pl.ds(base, SBN), :]
        vb = vs[pl.ds(base, SBN), :]
        s = _dotT(qb, kb) * scale
        m = jnp.max(s, axis=1, keepdims=True)
        e = jnp.exp(s - m) * bm
        a = e / jnp.sum(e, axis=1, keepdims=True)
        a1s[i, :, :] = a
        hn[pl.ds(base, SBN), :] = _dot(a, vb)
        return c

    jax.lax.fori_loop(0, NBLK, l1, 0, unroll=8)

    h1 = hn[:, :]
    qs[:, :] = _dot(h1, gw[3, :, :])
    ks[:, :] = _dot(h1, gw[4, :, :])
    vs[:, :] = _dot(h1, gw[5, :, :])

    def l2(i, c):
        base = pl.multiple_of(i * SBN, 8)
        qb = qs[pl.ds(base, SBN), :]
        kb = ks[pl.ds(base, SBN), :]
        vb = vs[pl.ds(base, SBN), :]
        s = _dotT(qb, kb) * scale * a1s[i, :, :]
        m = jnp.max(s, axis=1, keepdims=True)
        e = jnp.exp(s - m) * bm
        a = e / jnp.sum(e, axis=1, keepdims=True)
        for j in range(SB):
            a2s[i * SB + j, :, :] = a[j * NS:(j + 1) * NS,
                                      j * NS:(j + 1) * NS]
        hout[pl.ds(base, SBN), :] = _dot(a, vb)
        return c

    jax.lax.fori_loop(0, NBLK, l2, 0, unroll=8)

    amat = a2s[:, :, :]

    def dloop(i, tot):
        bi = a2s[pl.ds(i, 1), :, :]
        diff = amat - bi
        s2 = jnp.sum(diff * diff, axis=2)
        sj = jnp.sum(s2, axis=1, keepdims=True)
        return tot + jnp.sum(jnp.sqrt(sj + 1e-12))

    tot = jax.lax.fori_loop(0, B, dloop, jnp.zeros((1, 1), _F32), unroll=8)
    dist[:, :] = tot / float(B * B)


def _enc_body(xseq, scst, valid, msel, stat, wstat, bstat,
              ew, eb, ln, f1w, f1b, f2w, f2b,
              wm1, bm1, wm2, bm2,
              logits, xs, qs, ks, vs, os_, ffs, aggs):
    xs[:, :DM] = xseq[:, :]
    sc = scst[:, :]
    xs[:, DM:DM + DPE // 2] = jnp.sin(sc)
    xs[:, DM + DPE // 2:DTR] = jnp.cos(sc)

    hscale = 1.0 / math.sqrt(float(DH))
    bmt = _blockdiag_mask(TP)

    for l in range(2):
        x = xs[:, :]
        qs[:, :] = _dot(x, ew[4 * l + 0, :, :]) + eb[4 * l + 0:4 * l + 1, :]
        ks[:, :] = _dot(x, ew[4 * l + 1, :, :]) + eb[4 * l + 1:4 * l + 2, :]
        vs[:, :] = _dot(x, ew[4 * l + 2, :, :]) + eb[4 * l + 2:4 * l + 3, :]

        def attn_b(i, c):
            base = pl.multiple_of(i * SB * TP, 8)
            vcol = valid[pl.ds(base, SB * TP), :]
            for hh in range(NH):
                lo = hh * DH
                qh = qs[pl.ds(base, SB * TP), lo:lo + DH]
                kh = ks[pl.ds(base, SB * TP), lo:lo + DH]
                vh = vs[pl.ds(base, SB * TP), lo:lo + DH]
                sT = _dotT(kh, qh) * hscale
                m = jnp.max(sT, axis=0, keepdims=True)
                e = jnp.exp(sT - m) * (vcol * bmt)
                a = e / jnp.sum(e, axis=0, keepdims=True)
                os_[pl.ds(base, SB * TP), lo:lo + DH] = _dotC0(a, vh)
            return c

        jax.lax.fori_loop(0, NBLK, attn_b, 0, unroll=8)

        attn = _dot(os_[:, :], ew[4 * l + 3, :, :]) + eb[4 * l + 3:4 * l + 4, :]
        y = xs[:, :] + attn
        mu = jnp.mean(y, axis=1, keepdims=True)
        var = jnp.mean((y - mu) * (y - mu), axis=1, keepdims=True)
        yn = ((y - mu) * jax.lax.rsqrt(var + 1e-5) * ln[4 * l + 0:4 * l + 1, :]
              + ln[4 * l + 1:4 * l + 2, :])
        ffs[:, :] = jnp.maximum(_dot(yn, f1w[l, :, :]) + f1b[l:l + 1, :], 0.0)
        y2 = yn + _dot(ffs[:, :], f2w[l, :, :]) + f2b[l:l + 1, :]
        mu2 = jnp.mean(y2, axis=1, keepdims=True)
        var2 = jnp.mean((y2 - mu2) * (y2 - mu2), axis=1, keepdims=True)
        xs[:, :] = ((y2 - mu2) * jax.lax.rsqrt(var2 + 1e-5)
                    * ln[4 * l + 2:4 * l + 3, :] + ln[4 * l + 3:4 * l + 4, :])

    aggs[:, :DTR] = _dot(msel[:, :], xs[:, :])
    aggs[:, DTR:DFIN] = _dot(stat[:, :], wstat[:, :]) + bstat[:, :]
    hfin = jnp.maximum(_dot(aggs[:, :], wm1[:, :]) + bm1[:, :], 0.0)
    logits[:, :] = _dot(hfin, wm2[:, :]) + bm2[:, :]


def kernel(src, static, times, lengths, params):
    p = params
    f32 = _F32

    x = src[:, :, :NS]
    xt2d = jnp.transpose(x, (1, 2, 0)).reshape(BN, T)
    ru = jnp.broadcast_to(p['R_u'].reshape(NS, 1, DOB),
                          (NS, T, DOB)).reshape(NS, DF)
    ru4 = jnp.tile(ru, (SB, 1))
    ct = jnp.arange(DF)
    emat = (ct[None, :] // DOB == jnp.arange(T)[:, None]).astype(f32)
    gw = jnp.stack([p['Wq1'], p['Wk1'], p['Wv1'],
                    p['Wq2'], p['Wk2'], p['Wv2']])

    hout, dist = pl.pallas_call(
        _gat_body,
        out_shape=(jax.ShapeDtypeStruct((BN, DF), f32),
                   jax.ShapeDtypeStruct((1, 1), f32)),
        scratch_shapes=[
            pltpu.VMEM((BN, DF), f32),
            pltpu.VMEM((BN, DF), f32),
            pltpu.VMEM((BN, DF), f32),
            pltpu.VMEM((BN, DF), f32),
            pltpu.VMEM((NBLK, SBN, SBN), f32),
            pltpu.VMEM((B, NS, NS), f32),
        ],
    )(xt2d, emat, ru4, gw)

    seq = hout.reshape(B, NS, T, DOB).transpose(0, 2, 1, 3).reshape(B, T, DM)
    seq = jnp.pad(seq, ((0, 0), (0, TP - T), (0, 0))).reshape(BTP, DM)

    tpad = jnp.pad(times.T, ((0, 0), (0, TP - T)))
    scales = (MAXC ** jnp.linspace(0.0, 1.0, DPE // 2)).reshape(
        1, DPE // 2).astype(f32)
    scst = tpad.reshape(BTP, 1) / scales
    ar = jnp.arange(TP)
    validm = (ar[None, :] < lengths[:, None]).astype(f32)
    valid = validm.reshape(BTP, 1)
    lf = lengths[:, None].astype(f32)
    keep = validm / (lf + 1.0)
    msel = jax.lax.broadcasted_iota(jnp.int32, (B, B), 0)
    msel = (msel == jax.lax.broadcasted_iota(jnp.int32, (B, B), 1)).astype(f32)
    msel = (msel[:, :, None] * keep[None, :, :]).reshape(B, BTP)

    def r2(v):
        return v.reshape(1, -1)

    ew = jnp.stack([
        w for l in range(2) for w in (
            p['aWq%d' % l], p['aWk%d' % l], p['aWv%d' % l], p['aWo%d' % l])])
    eb = jnp.stack([
        v for l in range(2) for v in (
            p['abq%d' % l], p['abk%d' % l], p['abv%d' % l], p['abo%d' % l])])
    ln = jnp.stack([
        v for l in range(2) for v in (
            p['ln1s%d' % l], p['ln1b%d' % l],
            p['ln2s%d' % l], p['ln2b%d' % l])])
    f1w = jnp.stack([p['fW10'], p['fW11']])
    f1b = jnp.stack([p['fb10'], p['fb11']])
    f2w = jnp.stack([p['fW20'], p['fW21']])
    f2b = jnp.stack([p['fb20'], p['fb21']])
    ops = [seq, scst, valid, msel, static, p['W_static'], r2(p['b_static']),
           ew, eb, ln, f1w, f1b, f2w, f2b,
           p['Wm1'], r2(p['bm1']), p['Wm2'], r2(p['bm2'])]

    logits = pl.pallas_call(
        _enc_body,
        out_shape=jax.ShapeDtypeStruct((B, NCLS), f32),
        scratch_shapes=[
            pltpu.VMEM((BTP, DTR), f32),
            pltpu.VMEM((BTP, DTR), f32),
            pltpu.VMEM((BTP, DTR), f32),
            pltpu.VMEM((BTP, DTR), f32),
            pltpu.VMEM((BTP, DTR), f32),
            pltpu.VMEM((BTP, NHID), f32),
            pltpu.VMEM((B, DFIN), f32),
        ],
    )(*ops)

    return logits, dist[0, 0]
